# baseline (speedup 1.0000x reference)
import functools

import jax
import jax.numpy as jnp
from jax.experimental import pallas as pl
from jax.experimental.pallas import tpu as pltpu

_SPARSE_RATIO = 0.5
_CL = 512


def _body(x_ref, ltri_ref, o_ref, nb_ref, sq_ref, nbcol_ref, sc_ref,
          *, K: int, L: int):
    nc = L // _CL
    rows = _CL // 128
    c = pl.program_id(1)

    @pl.when(c == 0)
    def _per_row():
        for i in range(nc):
            part = x_ref[0, pl.ds(i * _CL, _CL), :]
            s = jnp.sum(part * part, axis=-1)
            nbcol_ref[pl.ds(i * _CL, _CL), :] = (
                jax.lax.bitcast_convert_type(s, jnp.int32)[:, None])
            s2 = s.reshape(rows, 128)
            nb_ref[pl.ds(i * rows, rows), :] = (
                jax.lax.bitcast_convert_type(jnp.sqrt(s2), jnp.int32))
            sq_ref[pl.ds(i * rows, rows), :] = (
                jax.lax.bitcast_convert_type(s2, jnp.int32))

        bits2 = nb_ref[...]

        def step(i, t):
            cand = jnp.bitwise_or(t, jnp.left_shift(jnp.int32(1), 30 - i))
            cnt = jnp.sum(jnp.where(bits2 >= cand, 1, 0))
            return jnp.where(cnt >= K, cand, t)

        T = jax.lax.fori_loop(0, 31, step, jnp.int32(0))
        eq2 = bits2 == T
        g = jnp.sum(jnp.where(bits2 > T, 1, 0))
        sq2 = sq_ref[...]
        sc_ref[0] = jnp.min(jnp.where(eq2, sq2, jnp.int32(2147483647)))
        sc_ref[1] = jnp.max(jnp.where(eq2, sq2, jnp.int32(-1)))
        sc_ref[2] = K - g
        riota = jax.lax.broadcasted_iota(jnp.int32, bits2.shape, 0)
        for cc in range(nc):
            sc_ref[3 + cc] = jnp.sum(
                jnp.where(jnp.logical_and(eq2, riota < cc * rows), 1, 0))

    lo = sc_ref[0]
    hi = sc_ref[1]
    m = sc_ref[2]
    pre = sc_ref[3 + c]

    bc = nbcol_ref[pl.ds(c * _CL, _CL), :]
    gt = (bc > hi).astype(jnp.float32)
    eq = (jnp.logical_and(bc >= lo, bc <= hi)).astype(jnp.float32)
    pref = jnp.dot(ltri_ref[...], eq, preferred_element_type=jnp.float32)
    cum = pref + pre.astype(jnp.float32)
    keepcol = gt + eq * (cum <= m.astype(jnp.float32)).astype(jnp.float32)
    o_ref[0] = x_ref[0, pl.ds(c * _CL, _CL), :] * keepcol


def kernel(x):
    B, L, C = x.shape
    K = max(1, int(L * (1.0 - _SPARSE_RATIO)))
    nc = L // _CL

    ltri = jnp.tri(_CL, dtype=jnp.float32)

    return pl.pallas_call(
        functools.partial(_body, K=K, L=L),
        grid=(B, nc),
        in_specs=[
            pl.BlockSpec((1, L, C), lambda b, c: (b, 0, 0)),
            pl.BlockSpec((_CL, _CL), lambda b, c: (0, 0)),
        ],
        out_specs=pl.BlockSpec((1, _CL, C), lambda b, c: (b, c, 0)),
        out_shape=jax.ShapeDtypeStruct((B, L, C), x.dtype),
        scratch_shapes=[
            pltpu.VMEM((L // 128, 128), jnp.int32),
            pltpu.VMEM((L // 128, 128), jnp.int32),
            pltpu.VMEM((L, 1), jnp.int32),
            pltpu.SMEM((3 + nc,), jnp.int32),
        ],
    )(x, ltri)


# speedup vs baseline: 1.2199x; 1.2199x over previous
import functools

import jax
import jax.numpy as jnp
from jax.experimental import pallas as pl
from jax.experimental.pallas import tpu as pltpu

_SPARSE_RATIO = 0.5
_CL = 512


def _per_row_phase(buf, nb_ref, sq_ref, nbcol_ref, sc_ref, K, L):
    nc = L // _CL
    rows = _CL // 128
    for i in range(nc):
        part = buf[pl.ds(i * _CL, _CL), :]
        s = jnp.sum(part * part, axis=-1)
        nbcol_ref[pl.ds(i * _CL, _CL), :] = (
            jax.lax.bitcast_convert_type(s, jnp.int32)[:, None])
        s2 = s.reshape(rows, 128)
        nb_ref[pl.ds(i * rows, rows), :] = (
            jax.lax.bitcast_convert_type(jnp.sqrt(s2), jnp.int32))
        sq_ref[pl.ds(i * rows, rows), :] = (
            jax.lax.bitcast_convert_type(s2, jnp.int32))

    bits2 = nb_ref[...]

    def step(i, t):
        cand = jnp.bitwise_or(t, jnp.left_shift(jnp.int32(1), 30 - i))
        cnt = jnp.sum(jnp.where(bits2 >= cand, 1, 0))
        return jnp.where(cnt >= K, cand, t)

    T = jax.lax.fori_loop(0, 31, step, jnp.int32(0))
    eq2 = bits2 == T
    g = jnp.sum(jnp.where(bits2 > T, 1, 0))
    sq2 = sq_ref[...]
    sc_ref[0] = jnp.min(jnp.where(eq2, sq2, jnp.int32(2147483647)))
    sc_ref[1] = jnp.max(jnp.where(eq2, sq2, jnp.int32(-1)))
    sc_ref[2] = K - g
    riota = jax.lax.broadcasted_iota(jnp.int32, bits2.shape, 0)
    for cc in range(nc):
        sc_ref[3 + cc] = jnp.sum(
            jnp.where(jnp.logical_and(eq2, riota < cc * rows), 1, 0))


def _body(x_hbm, ltri_ref, o_ref, buf0, buf1, nb_ref, sq_ref, nbcol_ref,
          sc_ref, sem0, sem1, *, K: int, L: int, B: int):
    nc = L // _CL
    c = pl.program_id(1)
    b = pl.program_id(0)

    @pl.when(c == 0)
    def _row_setup():
        @pl.when(b == 0)
        def _():
            pltpu.make_async_copy(x_hbm.at[0], buf0, sem0).start()

        @pl.when(b % 2 == 0)
        def _():
            pltpu.make_async_copy(x_hbm.at[b], buf0, sem0).wait()

        @pl.when(b % 2 == 1)
        def _():
            pltpu.make_async_copy(x_hbm.at[b], buf1, sem1).wait()

        @pl.when(jnp.logical_and(b + 1 < B, (b + 1) % 2 == 0))
        def _():
            pltpu.make_async_copy(x_hbm.at[b + 1], buf0, sem0).start()

        @pl.when(jnp.logical_and(b + 1 < B, (b + 1) % 2 == 1))
        def _():
            pltpu.make_async_copy(x_hbm.at[b + 1], buf1, sem1).start()

        @pl.when(b % 2 == 0)
        def _():
            _per_row_phase(buf0, nb_ref, sq_ref, nbcol_ref, sc_ref, K, L)

        @pl.when(b % 2 == 1)
        def _():
            _per_row_phase(buf1, nb_ref, sq_ref, nbcol_ref, sc_ref, K, L)

    lo = sc_ref[0]
    hi = sc_ref[1]
    m = sc_ref[2]
    pre = sc_ref[3 + c]

    bc = nbcol_ref[pl.ds(c * _CL, _CL), :]
    gt = (bc > hi).astype(jnp.float32)
    eq = (jnp.logical_and(bc >= lo, bc <= hi)).astype(jnp.float32)
    pref = jnp.dot(ltri_ref[...], eq, preferred_element_type=jnp.float32)
    cum = pref + pre.astype(jnp.float32)
    keepcol = gt + eq * (cum <= m.astype(jnp.float32)).astype(jnp.float32)

    @pl.when(b % 2 == 0)
    def _():
        o_ref[0] = buf0[pl.ds(c * _CL, _CL), :] * keepcol

    @pl.when(b % 2 == 1)
    def _():
        o_ref[0] = buf1[pl.ds(c * _CL, _CL), :] * keepcol


def kernel(x):
    B, L, C = x.shape
    K = max(1, int(L * (1.0 - _SPARSE_RATIO)))
    nc = L // _CL

    ltri = jnp.tri(_CL, dtype=jnp.float32)

    return pl.pallas_call(
        functools.partial(_body, K=K, L=L, B=B),
        grid=(B, nc),
        in_specs=[
            pl.BlockSpec(memory_space=pl.ANY),
            pl.BlockSpec((_CL, _CL), lambda b, c: (0, 0)),
        ],
        out_specs=pl.BlockSpec((1, _CL, C), lambda b, c: (b, c, 0)),
        out_shape=jax.ShapeDtypeStruct((B, L, C), x.dtype),
        scratch_shapes=[
            pltpu.VMEM((L, C), jnp.float32),
            pltpu.VMEM((L, C), jnp.float32),
            pltpu.VMEM((L // 128, 128), jnp.int32),
            pltpu.VMEM((L // 128, 128), jnp.int32),
            pltpu.VMEM((L, 1), jnp.int32),
            pltpu.SMEM((3 + nc,), jnp.int32),
            pltpu.SemaphoreType.DMA,
            pltpu.SemaphoreType.DMA,
        ],
    )(x, ltri)
